# Initial kernel scaffold; baseline (speedup 1.0000x reference)
import jax
import jax.numpy as jnp
from jax.experimental import pallas as pl


def kernel(coarse, fine, gt, alpha):
    raise NotImplementedError("write your pallas kernel here")


# speedup vs baseline: 23.7851x; 23.7851x over previous
import functools

import jax
import jax.numpy as jnp
from jax.experimental import pallas as pl
from jax.experimental.pallas import tpu as pltpu

B = 8
K = 1024
NF = 2048
NG = 4096
NEG = -1e10
HALF_NEG = -5e9
EPS = 0.005
ITERS = 50
SR = 64
NSTRIP = K // SR


def _fps_body(gt_ref, sel_ref, dists_ref):
    gx = gt_ref[0]
    gy = gt_ref[1]
    gz = gt_ref[2]
    lane = jax.lax.broadcasted_iota(jnp.int32, (B, NG), 1)

    dists_ref[...] = jnp.full((B, NG), 1e10, jnp.float32)
    sel_ref[...] = jnp.full((B, NG), K, jnp.int32)

    def step(k, last):
        mask2 = lane == last
        sel_ref[...] = jnp.where(mask2, k, sel_ref[...])
        lx = jnp.sum(jnp.where(mask2, gx, 0.0), axis=1, keepdims=True)
        ly = jnp.sum(jnp.where(mask2, gy, 0.0), axis=1, keepdims=True)
        lz = jnp.sum(jnp.where(mask2, gz, 0.0), axis=1, keepdims=True)
        dx = gx - lx
        dy = gy - ly
        dz = gz - lz
        d = (dx * dx + dy * dy) + dz * dz
        dists = jnp.minimum(dists_ref[...], d)
        dists_ref[...] = dists
        m = jnp.max(dists, axis=1, keepdims=True)
        nxt = jnp.min(jnp.where(dists == m, lane, NG), axis=1, keepdims=True)
        return nxt

    jax.lax.fori_loop(0, K, step, jnp.zeros((B, 1), jnp.int32))


def _run_fps(gt3):
    return pl.pallas_call(
        _fps_body,
        out_shape=jax.ShapeDtypeStruct((B, NG), jnp.int32),
        scratch_shapes=[pltpu.VMEM((B, NG), jnp.float32)],
    )(gt3)


def _emd_body(coarse_ref, y_ref, out_ref, ncost_ref, bj_ref, bidm_ref,
              price_ref, owner_ref):
    gx = y_ref[0:1, :]
    gy = y_ref[1:2, :]
    gz = y_ref[2:3, :]
    colj = jax.lax.broadcasted_iota(jnp.int32, (SR, K), 1)
    rowi = jax.lax.broadcasted_iota(jnp.int32, (SR, 1), 0)

    for s in range(NSTRIP):
        r = slice(s * SR, (s + 1) * SR)
        cx = coarse_ref[r, 0:1]
        cy = coarse_ref[r, 1:2]
        cz = coarse_ref[r, 2:3]
        dx = cx - gx
        dy = cy - gy
        dz = cz - gz
        ncost_ref[r, :] = -((dx * dx + dy * dy) + dz * dz)
    price_ref[...] = jnp.zeros((1, K), jnp.float32)
    owner_ref[...] = jnp.full((1, K), -1, jnp.int32)

    def auction_iter(_, carry):
        price = price_ref[...]
        owner = owner_ref[...]
        high = jnp.full((1, K), NEG, jnp.float32)
        for s in range(NSTRIP):
            r = slice(s * SR, (s + 1) * SR)
            rid = rowi + (s * SR)
            unass = ~jnp.any(owner == rid, axis=1, keepdims=True)
            value = ncost_ref[r, :] - price
            m1 = jnp.max(value, axis=1, keepdims=True)
            i1 = jnp.min(jnp.where(value == m1, colj, K), axis=1,
                         keepdims=True)
            mask = colj == i1
            v2 = jnp.max(jnp.where(mask, NEG, value), axis=1, keepdims=True)
            p_sel = jnp.max(jnp.where(mask, price, NEG), axis=1,
                            keepdims=True)
            bid = (p_sel + (m1 - v2)) + EPS
            bid_m = jnp.where(unass, bid, NEG)
            bj_ref[r, :] = i1
            bidm_ref[r, :] = bid_m
            contrib = jnp.where(mask, bid_m, NEG)
            high = jnp.maximum(high, jnp.max(contrib, axis=0, keepdims=True))
        winner = jnp.full((1, K), K, jnp.int32)
        for s in range(NSTRIP):
            r = slice(s * SR, (s + 1) * SR)
            i1 = bj_ref[r, :]
            bid_m = bidm_ref[r, :]
            ok = (colj == i1) & (bid_m == high) & (bid_m > HALF_NEG)
            cand = jnp.where(ok, rowi + (s * SR), K)
            winner = jnp.minimum(winner, jnp.min(cand, axis=0, keepdims=True))
        has_w = winner < K
        wc = jnp.minimum(winner, K - 1)
        price_ref[...] = jnp.where(has_w, high, price)
        owner_ref[...] = jnp.where(has_w, wc, owner)
        return carry

    jax.lax.fori_loop(0, ITERS, auction_iter, jnp.int32(0))

    owner = owner_ref[...]
    acc = jnp.zeros((1, 1), jnp.float32)
    for s in range(NSTRIP):
        r = slice(s * SR, (s + 1) * SR)
        rid = rowi + (s * SR)
        a = jnp.min(jnp.where(owner == rid, colj, K), axis=1, keepdims=True)
        nc = ncost_ref[r, :]
        cmax = jnp.max(nc, axis=1, keepdims=True)
        jmin = jnp.min(jnp.where(nc == cmax, colj, K), axis=1, keepdims=True)
        a = jnp.where(a == K, jmin, a)
        mask = colj == a
        mx = jnp.sum(jnp.where(mask, gx, 0.0), axis=1, keepdims=True)
        my = jnp.sum(jnp.where(mask, gy, 0.0), axis=1, keepdims=True)
        mz = jnp.sum(jnp.where(mask, gz, 0.0), axis=1, keepdims=True)
        cx = coarse_ref[r, 0:1]
        cy = coarse_ref[r, 1:2]
        cz = coarse_ref[r, 2:3]
        dx = cx - mx
        dy = cy - my
        dz = cz - mz
        dist = (dx * dx + dy * dy) + dz * dz
        acc = acc + jnp.sum(jnp.sqrt(jnp.maximum(dist, 1e-12)))
    out_ref[...] = acc


def _run_emd(coarse, gt_ds3):
    return pl.pallas_call(
        _emd_body,
        grid=(B,),
        in_specs=[
            pl.BlockSpec((None, K, 3), lambda b: (b, 0, 0)),
            pl.BlockSpec((None, 3, K), lambda b: (b, 0, 0)),
        ],
        out_specs=pl.BlockSpec((None, 1, 1), lambda b: (b, 0, 0)),
        out_shape=jax.ShapeDtypeStruct((B, 1, 1), jnp.float32),
        scratch_shapes=[
            pltpu.VMEM((K, K), jnp.float32),
            pltpu.VMEM((K, 1), jnp.int32),
            pltpu.VMEM((K, 1), jnp.float32),
            pltpu.VMEM((1, K), jnp.float32),
            pltpu.VMEM((1, K), jnp.int32),
        ],
    )(coarse, gt_ds3)


CSR = 16
NCSTRIP = NF // CSR


def _chamfer_body(fine_ref, gt_ref, out_ref):
    gx = gt_ref[0:1, :]
    gy = gt_ref[1:2, :]
    gz = gt_ref[2:3, :]
    cm = jnp.full((1, NG), 1e30, jnp.float32)
    acc = jnp.zeros((1, 1), jnp.float32)
    for s in range(NCSTRIP):
        r = slice(s * CSR, (s + 1) * CSR)
        fx = fine_ref[r, 0:1]
        fy = fine_ref[r, 1:2]
        fz = fine_ref[r, 2:3]
        dx = fx - gx
        dy = fy - gy
        dz = fz - gz
        d2 = (dx * dx + dy * dy) + dz * dz
        acc = acc + jnp.sum(jnp.min(d2, axis=1))
        cm = jnp.minimum(cm, jnp.min(d2, axis=0, keepdims=True))
    out_ref[...] = acc / NF + jnp.sum(cm) / NG


def _run_chamfer(fine, gt3):
    return pl.pallas_call(
        _chamfer_body,
        grid=(B,),
        in_specs=[
            pl.BlockSpec((None, NF, 3), lambda b: (b, 0, 0)),
            pl.BlockSpec((None, 3, NG), lambda b: (b, 0, 0)),
        ],
        out_specs=pl.BlockSpec((None, 1, 1), lambda b: (b, 0, 0)),
        out_shape=jax.ShapeDtypeStruct((B, 1, 1), jnp.float32),
    )(fine, gt3)


def kernel(coarse, fine, gt, alpha):
    gt3 = jnp.transpose(gt, (1, 0, 2))
    sel = _run_fps(gt3)

    valid = sel < K
    sidx = jnp.where(valid, sel, K)
    bidx = jnp.arange(B, dtype=jnp.int32)[:, None]
    gt_dsb = jnp.zeros((B, 3, K), jnp.float32)
    for c in range(3):
        gt_dsb = gt_dsb.at[bidx, c, sidx].set(gt3[c], mode='drop')

    sums = _run_emd(coarse, gt_dsb)
    cham = _run_chamfer(fine, gt)

    loss_coarse = jnp.sum(sums) / (B * K)
    loss_fine = jnp.mean(cham)
    loss = loss_coarse + alpha * loss_fine
    return (loss, loss_coarse, loss_fine)


# speedup vs baseline: 26.1828x; 1.1008x over previous
import functools

import jax
import jax.numpy as jnp
from jax import lax
from jax.experimental import pallas as pl
from jax.experimental.pallas import tpu as pltpu
from jax.experimental.pallas import tpu_sc as plsc

B = 8
K = 1024
NF = 2048
NG = 4096
NEG = -1e10
HALF_NEG = -5e9
EPS = 0.005
ITERS = 50
SR = 64
NSTRIP = K // SR


def _fps_body(gt_ref, sel_ref, dists_ref):
    gx = gt_ref[0]
    gy = gt_ref[1]
    gz = gt_ref[2]
    lane = jax.lax.broadcasted_iota(jnp.int32, (B, NG), 1)

    dists_ref[...] = jnp.full((B, NG), 1e10, jnp.float32)
    sel_ref[...] = jnp.full((B, NG), K, jnp.int32)

    def step(k, last):
        mask2 = lane == last
        sel_ref[...] = jnp.where(mask2, k, sel_ref[...])
        lx = jnp.sum(jnp.where(mask2, gx, 0.0), axis=1, keepdims=True)
        ly = jnp.sum(jnp.where(mask2, gy, 0.0), axis=1, keepdims=True)
        lz = jnp.sum(jnp.where(mask2, gz, 0.0), axis=1, keepdims=True)
        dx = gx - lx
        dy = gy - ly
        dz = gz - lz
        d = (dx * dx + dy * dy) + dz * dz
        dists = jnp.minimum(dists_ref[...], d)
        dists_ref[...] = dists
        m = jnp.max(dists, axis=1, keepdims=True)
        nxt = jnp.min(jnp.where(dists == m, lane, NG), axis=1, keepdims=True)
        return nxt

    jax.lax.fori_loop(0, K, step, jnp.zeros((B, 1), jnp.int32))


def _run_fps(gt3):
    return pl.pallas_call(
        _fps_body,
        out_shape=jax.ShapeDtypeStruct((B, NG), jnp.int32),
        scratch_shapes=[pltpu.VMEM((B, NG), jnp.float32)],
    )(gt3)


KP = K + 8
NCHUNK = NG // 128


def _run_scatter_sc(gt_pad, sel3):
    info = plsc.get_sparse_core_info()
    nc, ns, nl = info.num_cores, info.num_subcores, info.num_lanes

    @functools.partial(
        pl.kernel,
        mesh=plsc.VectorSubcoreMesh(core_axis_name="c", subcore_axis_name="s"),
        out_type=jax.ShapeDtypeStruct((B * KP, 128), jnp.float32),
        scratch_types=[
            pltpu.VMEM((NCHUNK, 128), jnp.int32),
            pltpu.VMEM((128, 128), jnp.float32),
            pltpu.SemaphoreType.DMA,
        ],
    )
    def k(gt_hbm, sel_hbm, out_hbm, idx_v, rows_v, sem):
        wid = lax.axis_index("s") * nc + lax.axis_index("c")

        @pl.when(wid < B)
        def _():
            b = wid
            pltpu.sync_copy(sel_hbm.at[b], idx_v)
            off = b * KP

            def obody(i, carry):
                r = i // (128 // nl)
                g = i % (128 // nl)
                s = pl.ds(g * nl, nl)
                idx_v[r, s] = idx_v[r, s] + off
                return carry

            lax.fori_loop(0, NCHUNK * (128 // nl), obody, jnp.int32(0))

            def body(j, carry):
                pltpu.sync_copy(gt_hbm.at[b, pl.ds(j * 128, 128)], rows_v)
                pltpu.async_copy(
                    rows_v,
                    out_hbm.at[idx_v.at[j]],
                    sem,
                ).wait()
                return carry

            lax.fori_loop(0, NCHUNK, body, jnp.int32(0))

    return k(gt_pad, sel3)


def _emd_body(coarse_ref, y_ref, out_ref, ncost_ref, bj_ref, bidm_ref,
              price_ref, owner_ref):
    gx = y_ref[0:1, :]
    gy = y_ref[1:2, :]
    gz = y_ref[2:3, :]
    colj = jax.lax.broadcasted_iota(jnp.int32, (SR, K), 1)
    rowi = jax.lax.broadcasted_iota(jnp.int32, (SR, 1), 0)

    for s in range(NSTRIP):
        r = slice(s * SR, (s + 1) * SR)
        cx = coarse_ref[r, 0:1]
        cy = coarse_ref[r, 1:2]
        cz = coarse_ref[r, 2:3]
        dx = cx - gx
        dy = cy - gy
        dz = cz - gz
        ncost_ref[r, :] = -((dx * dx + dy * dy) + dz * dz)
    price_ref[...] = jnp.zeros((1, K), jnp.float32)
    owner_ref[...] = jnp.full((1, K), -1, jnp.int32)

    def auction_iter(_, carry):
        price = price_ref[...]
        owner = owner_ref[...]
        high = jnp.full((1, K), NEG, jnp.float32)
        for s in range(NSTRIP):
            r = slice(s * SR, (s + 1) * SR)
            rid = rowi + (s * SR)
            unass = ~jnp.any(owner == rid, axis=1, keepdims=True)
            value = ncost_ref[r, :] - price
            m1 = jnp.max(value, axis=1, keepdims=True)
            i1 = jnp.min(jnp.where(value == m1, colj, K), axis=1,
                         keepdims=True)
            mask = colj == i1
            v2 = jnp.max(jnp.where(mask, NEG, value), axis=1, keepdims=True)
            p_sel = jnp.max(jnp.where(mask, price, NEG), axis=1,
                            keepdims=True)
            bid = (p_sel + (m1 - v2)) + EPS
            bid_m = jnp.where(unass, bid, NEG)
            bj_ref[r, :] = i1
            bidm_ref[r, :] = bid_m
            contrib = jnp.where(mask, bid_m, NEG)
            high = jnp.maximum(high, jnp.max(contrib, axis=0, keepdims=True))
        winner = jnp.full((1, K), K, jnp.int32)
        for s in range(NSTRIP):
            r = slice(s * SR, (s + 1) * SR)
            i1 = bj_ref[r, :]
            bid_m = bidm_ref[r, :]
            ok = (colj == i1) & (bid_m == high) & (bid_m > HALF_NEG)
            cand = jnp.where(ok, rowi + (s * SR), K)
            winner = jnp.minimum(winner, jnp.min(cand, axis=0, keepdims=True))
        has_w = winner < K
        wc = jnp.minimum(winner, K - 1)
        price_ref[...] = jnp.where(has_w, high, price)
        owner_ref[...] = jnp.where(has_w, wc, owner)
        return carry

    jax.lax.fori_loop(0, ITERS, auction_iter, jnp.int32(0))

    owner = owner_ref[...]
    acc = jnp.zeros((1, 1), jnp.float32)
    for s in range(NSTRIP):
        r = slice(s * SR, (s + 1) * SR)
        rid = rowi + (s * SR)
        a = jnp.min(jnp.where(owner == rid, colj, K), axis=1, keepdims=True)
        nc = ncost_ref[r, :]
        cmax = jnp.max(nc, axis=1, keepdims=True)
        jmin = jnp.min(jnp.where(nc == cmax, colj, K), axis=1, keepdims=True)
        a = jnp.where(a == K, jmin, a)
        mask = colj == a
        mx = jnp.sum(jnp.where(mask, gx, 0.0), axis=1, keepdims=True)
        my = jnp.sum(jnp.where(mask, gy, 0.0), axis=1, keepdims=True)
        mz = jnp.sum(jnp.where(mask, gz, 0.0), axis=1, keepdims=True)
        cx = coarse_ref[r, 0:1]
        cy = coarse_ref[r, 1:2]
        cz = coarse_ref[r, 2:3]
        dx = cx - mx
        dy = cy - my
        dz = cz - mz
        dist = (dx * dx + dy * dy) + dz * dz
        acc = acc + jnp.sum(jnp.sqrt(jnp.maximum(dist, 1e-12)))
    out_ref[...] = acc


def _run_emd(coarse, gt_ds3):
    return pl.pallas_call(
        _emd_body,
        grid=(B,),
        in_specs=[
            pl.BlockSpec((None, K, 3), lambda b: (b, 0, 0)),
            pl.BlockSpec((None, 3, K), lambda b: (b, 0, 0)),
        ],
        out_specs=pl.BlockSpec((None, 1, 1), lambda b: (b, 0, 0)),
        out_shape=jax.ShapeDtypeStruct((B, 1, 1), jnp.float32),
        scratch_shapes=[
            pltpu.VMEM((K, K), jnp.float32),
            pltpu.VMEM((K, 1), jnp.int32),
            pltpu.VMEM((K, 1), jnp.float32),
            pltpu.VMEM((1, K), jnp.float32),
            pltpu.VMEM((1, K), jnp.int32),
        ],
    )(coarse, gt_ds3)


CSR = 16
NCSTRIP = NF // CSR


def _chamfer_body(fine_ref, gt_ref, out_ref):
    gx = gt_ref[0:1, :]
    gy = gt_ref[1:2, :]
    gz = gt_ref[2:3, :]
    cm = jnp.full((1, NG), 1e30, jnp.float32)
    acc = jnp.zeros((1, 1), jnp.float32)
    for s in range(NCSTRIP):
        r = slice(s * CSR, (s + 1) * CSR)
        fx = fine_ref[r, 0:1]
        fy = fine_ref[r, 1:2]
        fz = fine_ref[r, 2:3]
        dx = fx - gx
        dy = fy - gy
        dz = fz - gz
        d2 = (dx * dx + dy * dy) + dz * dz
        acc = acc + jnp.sum(jnp.min(d2, axis=1))
        cm = jnp.minimum(cm, jnp.min(d2, axis=0, keepdims=True))
    out_ref[...] = acc / NF + jnp.sum(cm) / NG


def _run_chamfer(fine, gt3):
    return pl.pallas_call(
        _chamfer_body,
        grid=(B,),
        in_specs=[
            pl.BlockSpec((None, NF, 3), lambda b: (b, 0, 0)),
            pl.BlockSpec((None, 3, NG), lambda b: (b, 0, 0)),
        ],
        out_specs=pl.BlockSpec((None, 1, 1), lambda b: (b, 0, 0)),
        out_shape=jax.ShapeDtypeStruct((B, 1, 1), jnp.float32),
    )(fine, gt3)


def kernel(coarse, fine, gt, alpha):
    gt3 = jnp.transpose(gt, (1, 0, 2))
    sel = _run_fps(gt3)

    gt_pad = jnp.pad(jnp.transpose(gt, (0, 2, 1)), ((0, 0), (0, 0), (0, 125)))
    sel3 = sel.reshape(B, NCHUNK, 128)
    flat = _run_scatter_sc(gt_pad, sel3)
    gt_dsb = jnp.transpose(
        flat.reshape(B, KP, 128)[:, :K, :3], (0, 2, 1))

    sums = _run_emd(coarse, gt_dsb)
    cham = _run_chamfer(fine, gt)

    loss_coarse = jnp.sum(sums) / (B * K)
    loss_fine = jnp.mean(cham)
    loss = loss_coarse + alpha * loss_fine
    return (loss, loss_coarse, loss_fine)


# speedup vs baseline: 26.1908x; 1.0003x over previous
import functools

import jax
import jax.numpy as jnp
from jax import lax
from jax.experimental import pallas as pl
from jax.experimental.pallas import tpu as pltpu
from jax.experimental.pallas import tpu_sc as plsc

B = 8
K = 1024
NF = 2048
NG = 4096
NEG = -1e10
HALF_NEG = -5e9
EPS = 0.005
ITERS = 50
SR = 64
NSTRIP = K // SR


def _fps_body(gt_ref, sel_ref, dists_ref):
    gx = gt_ref[0]
    gy = gt_ref[1]
    gz = gt_ref[2]
    lane = jax.lax.broadcasted_iota(jnp.int32, (B, NG), 1)

    dists_ref[...] = jnp.full((B, NG), 1e10, jnp.float32)
    sel_ref[...] = jnp.full((B, NG), K, jnp.int32)

    def step(k, last):
        mask2 = lane == last
        sel_ref[...] = jnp.where(mask2, k, sel_ref[...])
        lx = jnp.sum(jnp.where(mask2, gx, 0.0), axis=1, keepdims=True)
        ly = jnp.sum(jnp.where(mask2, gy, 0.0), axis=1, keepdims=True)
        lz = jnp.sum(jnp.where(mask2, gz, 0.0), axis=1, keepdims=True)
        dx = gx - lx
        dy = gy - ly
        dz = gz - lz
        d = (dx * dx + dy * dy) + dz * dz
        dists = jnp.minimum(dists_ref[...], d)
        dists_ref[...] = dists
        m = jnp.max(dists, axis=1, keepdims=True)
        nxt = jnp.min(jnp.where(dists == m, lane, NG), axis=1, keepdims=True)
        return nxt

    jax.lax.fori_loop(0, K, step, jnp.zeros((B, 1), jnp.int32))


def _run_fps(gt3):
    return pl.pallas_call(
        _fps_body,
        out_shape=jax.ShapeDtypeStruct((B, NG), jnp.int32),
        scratch_shapes=[pltpu.VMEM((B, NG), jnp.float32)],
    )(gt3)


KP = K + 8
NCHUNK = NG // 128


def _run_scatter_sc(gt_pad, sel3):
    info = plsc.get_sparse_core_info()
    nc, ns, nl = info.num_cores, info.num_subcores, info.num_lanes

    @functools.partial(
        pl.kernel,
        mesh=plsc.VectorSubcoreMesh(core_axis_name="c", subcore_axis_name="s"),
        out_type=jax.ShapeDtypeStruct((B * KP, 128), jnp.float32),
        scratch_types=[
            pltpu.VMEM((NCHUNK, 128), jnp.int32),
            pltpu.VMEM((128, 128), jnp.float32),
            pltpu.SemaphoreType.DMA,
        ],
    )
    def k(gt_hbm, sel_hbm, out_hbm, idx_v, rows_v, sem):
        wid = lax.axis_index("s") * nc + lax.axis_index("c")

        @pl.when(wid < B)
        def _():
            b = wid
            pltpu.sync_copy(sel_hbm.at[b], idx_v)
            off = b * KP

            def obody(i, carry):
                r = i // (128 // nl)
                g = i % (128 // nl)
                s = pl.ds(g * nl, nl)
                idx_v[r, s] = idx_v[r, s] + off
                return carry

            lax.fori_loop(0, NCHUNK * (128 // nl), obody, jnp.int32(0))

            def body(j, carry):
                pltpu.sync_copy(gt_hbm.at[b, pl.ds(j * 128, 128)], rows_v)
                pltpu.async_copy(
                    rows_v,
                    out_hbm.at[idx_v.at[j]],
                    sem,
                ).wait()
                return carry

            lax.fori_loop(0, NCHUNK, body, jnp.int32(0))

    return k(gt_pad, sel3)


def _emd_body(coarse_ref, y_ref, out_ref, ncost_ref, bj_ref, bidm_ref,
              price_ref, owner_ref):
    gx = y_ref[0:1, :]
    gy = y_ref[1:2, :]
    gz = y_ref[2:3, :]
    colj = jax.lax.broadcasted_iota(jnp.int32, (SR, K), 1)
    rowi = jax.lax.broadcasted_iota(jnp.int32, (SR, 1), 0)

    for s in range(NSTRIP):
        r = slice(s * SR, (s + 1) * SR)
        cx = coarse_ref[r, 0:1]
        cy = coarse_ref[r, 1:2]
        cz = coarse_ref[r, 2:3]
        dx = cx - gx
        dy = cy - gy
        dz = cz - gz
        ncost_ref[r, :] = -((dx * dx + dy * dy) + dz * dz)
    price_ref[...] = jnp.zeros((1, K), jnp.float32)
    owner_ref[...] = jnp.full((1, K), -1, jnp.int32)

    def auction_iter(_, carry):
        price = price_ref[...]
        owner = owner_ref[...]
        high = jnp.full((1, K), NEG, jnp.float32)
        for s in range(NSTRIP):
            r = slice(s * SR, (s + 1) * SR)
            rid = rowi + (s * SR)
            unass = ~jnp.any(owner == rid, axis=1, keepdims=True)
            value = ncost_ref[r, :] - price
            m1 = jnp.max(value, axis=1, keepdims=True)
            i1 = jnp.min(jnp.where(value == m1, colj, K), axis=1,
                         keepdims=True)
            mask = colj == i1
            v2 = jnp.max(jnp.where(mask, NEG, value), axis=1, keepdims=True)
            p_sel = jnp.max(jnp.where(mask, price, NEG), axis=1,
                            keepdims=True)
            bid = (p_sel + (m1 - v2)) + EPS
            bid_m = jnp.where(unass, bid, NEG)
            bj_ref[r, :] = i1
            bidm_ref[r, :] = bid_m
            contrib = jnp.where(mask, bid_m, NEG)
            high = jnp.maximum(high, jnp.max(contrib, axis=0, keepdims=True))
        winner = jnp.full((1, K), K, jnp.int32)
        for s in range(NSTRIP):
            r = slice(s * SR, (s + 1) * SR)
            i1 = bj_ref[r, :]
            bid_m = bidm_ref[r, :]
            ok = (colj == i1) & (bid_m == high) & (bid_m > HALF_NEG)
            cand = jnp.where(ok, rowi + (s * SR), K)
            winner = jnp.minimum(winner, jnp.min(cand, axis=0, keepdims=True))
        has_w = winner < K
        wc = jnp.minimum(winner, K - 1)
        price_ref[...] = jnp.where(has_w, high, price)
        owner_ref[...] = jnp.where(has_w, wc, owner)
        return carry

    jax.lax.fori_loop(0, ITERS, auction_iter, jnp.int32(0))

    owner = owner_ref[...]
    acc = jnp.zeros((1, 1), jnp.float32)
    for s in range(NSTRIP):
        r = slice(s * SR, (s + 1) * SR)
        rid = rowi + (s * SR)
        a = jnp.min(jnp.where(owner == rid, colj, K), axis=1, keepdims=True)
        nc = ncost_ref[r, :]
        cmax = jnp.max(nc, axis=1, keepdims=True)
        jmin = jnp.min(jnp.where(nc == cmax, colj, K), axis=1, keepdims=True)
        a = jnp.where(a == K, jmin, a)
        mask = colj == a
        mx = jnp.sum(jnp.where(mask, gx, 0.0), axis=1, keepdims=True)
        my = jnp.sum(jnp.where(mask, gy, 0.0), axis=1, keepdims=True)
        mz = jnp.sum(jnp.where(mask, gz, 0.0), axis=1, keepdims=True)
        cx = coarse_ref[r, 0:1]
        cy = coarse_ref[r, 1:2]
        cz = coarse_ref[r, 2:3]
        dx = cx - mx
        dy = cy - my
        dz = cz - mz
        dist = (dx * dx + dy * dy) + dz * dz
        acc = acc + jnp.sum(jnp.sqrt(jnp.maximum(dist, 1e-12)))
    out_ref[...] = acc


def _run_emd(coarse, gt_ds3):
    return pl.pallas_call(
        _emd_body,
        grid=(B,),
        in_specs=[
            pl.BlockSpec((None, K, 3), lambda b: (b, 0, 0)),
            pl.BlockSpec((None, 3, K), lambda b: (b, 0, 0)),
        ],
        out_specs=pl.BlockSpec((None, 1, 1), lambda b: (b, 0, 0)),
        out_shape=jax.ShapeDtypeStruct((B, 1, 1), jnp.float32),
        scratch_shapes=[
            pltpu.VMEM((K, K), jnp.float32),
            pltpu.VMEM((K, 1), jnp.int32),
            pltpu.VMEM((K, 1), jnp.float32),
            pltpu.VMEM((1, K), jnp.float32),
            pltpu.VMEM((1, K), jnp.int32),
        ],
        compiler_params=pltpu.CompilerParams(
            dimension_semantics=("parallel",)),
    )(coarse, gt_ds3)


CSR = 16
NCSTRIP = NF // CSR


def _chamfer_body(fine_ref, gt_ref, out_ref):
    gx = gt_ref[0:1, :]
    gy = gt_ref[1:2, :]
    gz = gt_ref[2:3, :]
    cm = jnp.full((1, NG), 1e30, jnp.float32)
    acc = jnp.zeros((1, 1), jnp.float32)
    for s in range(NCSTRIP):
        r = slice(s * CSR, (s + 1) * CSR)
        fx = fine_ref[r, 0:1]
        fy = fine_ref[r, 1:2]
        fz = fine_ref[r, 2:3]
        dx = fx - gx
        dy = fy - gy
        dz = fz - gz
        d2 = (dx * dx + dy * dy) + dz * dz
        acc = acc + jnp.sum(jnp.min(d2, axis=1))
        cm = jnp.minimum(cm, jnp.min(d2, axis=0, keepdims=True))
    out_ref[...] = acc / NF + jnp.sum(cm) / NG


def _run_chamfer(fine, gt3):
    return pl.pallas_call(
        _chamfer_body,
        grid=(B,),
        in_specs=[
            pl.BlockSpec((None, NF, 3), lambda b: (b, 0, 0)),
            pl.BlockSpec((None, 3, NG), lambda b: (b, 0, 0)),
        ],
        out_specs=pl.BlockSpec((None, 1, 1), lambda b: (b, 0, 0)),
        out_shape=jax.ShapeDtypeStruct((B, 1, 1), jnp.float32),
        compiler_params=pltpu.CompilerParams(
            dimension_semantics=("parallel",)),
    )(fine, gt3)


def kernel(coarse, fine, gt, alpha):
    gt3 = jnp.transpose(gt, (1, 0, 2))
    sel = _run_fps(gt3)

    gt_pad = jnp.pad(jnp.transpose(gt, (0, 2, 1)), ((0, 0), (0, 0), (0, 125)))
    sel3 = sel.reshape(B, NCHUNK, 128)
    flat = _run_scatter_sc(gt_pad, sel3)
    gt_dsb = jnp.transpose(
        flat.reshape(B, KP, 128)[:, :K, :3], (0, 2, 1))

    sums = _run_emd(coarse, gt_dsb)
    cham = _run_chamfer(fine, gt)

    loss_coarse = jnp.sum(sums) / (B * K)
    loss_fine = jnp.mean(cham)
    loss = loss_coarse + alpha * loss_fine
    return (loss, loss_coarse, loss_fine)


# speedup vs baseline: 34.2421x; 1.3074x over previous
import functools

import jax
import jax.numpy as jnp
from jax import lax
from jax.experimental import pallas as pl
from jax.experimental.pallas import tpu as pltpu
from jax.experimental.pallas import tpu_sc as plsc

B = 8
K = 1024
NF = 2048
NG = 4096
NEG = -1e10
HALF_NEG = -5e9
EPS = 0.005
ITERS = 50
SR = 64
NSTRIP = K // SR


def _fps_body(gt_ref, sel_ref, dists_ref):
    gx = gt_ref[0]
    gy = gt_ref[1]
    gz = gt_ref[2]
    lane = jax.lax.broadcasted_iota(jnp.int32, (B, NG), 1)

    dists_ref[...] = jnp.full((B, NG), 1e10, jnp.float32)
    sel_ref[...] = jnp.full((B, NG), K, jnp.int32)

    def step(k, last):
        mask2 = lane == last
        sel_ref[...] = jnp.where(mask2, k, sel_ref[...])
        lx = jnp.sum(jnp.where(mask2, gx, 0.0), axis=1, keepdims=True)
        ly = jnp.sum(jnp.where(mask2, gy, 0.0), axis=1, keepdims=True)
        lz = jnp.sum(jnp.where(mask2, gz, 0.0), axis=1, keepdims=True)
        dx = gx - lx
        dy = gy - ly
        dz = gz - lz
        d = (dx * dx + dy * dy) + dz * dz
        dists = jnp.minimum(dists_ref[...], d)
        dists_ref[...] = dists
        m = jnp.max(dists, axis=1, keepdims=True)
        nxt = jnp.min(jnp.where(dists == m, lane, NG), axis=1, keepdims=True)
        return nxt

    jax.lax.fori_loop(0, K, step, jnp.zeros((B, 1), jnp.int32))


def _run_fps(gt3):
    return pl.pallas_call(
        _fps_body,
        out_shape=jax.ShapeDtypeStruct((B, NG), jnp.int32),
        scratch_shapes=[pltpu.VMEM((B, NG), jnp.float32)],
    )(gt3)


KP = K + 8
NCHUNK = NG // 128


def _run_scatter_sc(gt_pad, sel3):
    info = plsc.get_sparse_core_info()
    nc, ns, nl = info.num_cores, info.num_subcores, info.num_lanes

    @functools.partial(
        pl.kernel,
        mesh=plsc.VectorSubcoreMesh(core_axis_name="c", subcore_axis_name="s"),
        out_type=jax.ShapeDtypeStruct((B * KP, 128), jnp.float32),
        scratch_types=[
            pltpu.VMEM((NCHUNK, 128), jnp.int32),
            pltpu.VMEM((128, 128), jnp.float32),
            pltpu.SemaphoreType.DMA,
        ],
    )
    def k(gt_hbm, sel_hbm, out_hbm, idx_v, rows_v, sem):
        wid = lax.axis_index("s") * nc + lax.axis_index("c")

        @pl.when(wid < B)
        def _():
            b = wid
            pltpu.sync_copy(sel_hbm.at[b], idx_v)
            off = b * KP

            def obody(i, carry):
                r = i // (128 // nl)
                g = i % (128 // nl)
                s = pl.ds(g * nl, nl)
                idx_v[r, s] = idx_v[r, s] + off
                return carry

            lax.fori_loop(0, NCHUNK * (128 // nl), obody, jnp.int32(0))

            def body(j, carry):
                pltpu.sync_copy(gt_hbm.at[b, pl.ds(j * 128, 128)], rows_v)
                pltpu.async_copy(
                    rows_v,
                    out_hbm.at[idx_v.at[j]],
                    sem,
                ).wait()
                return carry

            lax.fori_loop(0, NCHUNK, body, jnp.int32(0))

    return k(gt_pad, sel3)


def _emd_body(coarse_ref, y_ref, out_ref, ncost_ref, pmax_ref, parg_ref,
              price_ref, owner_ref):
    gx = y_ref[0:1, :]
    gy = y_ref[1:2, :]
    gz = y_ref[2:3, :]
    colj = jax.lax.broadcasted_iota(jnp.int32, (SR, K), 1)
    rowi = jax.lax.broadcasted_iota(jnp.int32, (SR, 1), 0)

    for s in range(NSTRIP):
        r = slice(s * SR, (s + 1) * SR)
        cx = coarse_ref[r, 0:1]
        cy = coarse_ref[r, 1:2]
        cz = coarse_ref[r, 2:3]
        dx = cx - gx
        dy = cy - gy
        dz = cz - gz
        ncost_ref[r, :] = -((dx * dx + dy * dy) + dz * dz)
    price_ref[...] = jnp.zeros((1, K), jnp.float32)
    owner_ref[...] = jnp.full((1, K), -1, jnp.int32)

    def auction_iter(_, carry):
        price = price_ref[...]
        owner = owner_ref[...]
        for s in range(NSTRIP):
            r = slice(s * SR, (s + 1) * SR)
            rid = rowi + (s * SR)
            unass = ~jnp.any(owner == rid, axis=1, keepdims=True)
            value = ncost_ref[r, :] - price
            m1 = jnp.max(value, axis=1, keepdims=True)
            i1 = jnp.min(jnp.where(value == m1, colj, K), axis=1,
                         keepdims=True)
            mask = colj == i1
            v2 = jnp.max(jnp.where(mask, NEG, value), axis=1, keepdims=True)
            bid_m = jnp.where(unass, (m1 - v2) + EPS, NEG)
            contrib = jnp.where(mask, bid_m, NEG)
            pm = jnp.max(contrib, axis=0, keepdims=True)
            pa = jnp.min(
                jnp.where((contrib == pm) & (pm > HALF_NEG), rid, K),
                axis=0, keepdims=True)
            pmax_ref[s:s + 1, :] = pm
            parg_ref[s:s + 1, :] = pa
        pms = pmax_ref[...]
        hd = jnp.max(pms, axis=0, keepdims=True)
        winner = jnp.min(
            jnp.where(pms == hd, parg_ref[...], K), axis=0, keepdims=True)
        has_w = winner < K
        wc = jnp.minimum(winner, K - 1)
        price_ref[...] = jnp.where(has_w, price + hd, price)
        owner_ref[...] = jnp.where(has_w, wc, owner)
        return carry

    jax.lax.fori_loop(0, ITERS, auction_iter, jnp.int32(0))

    owner = owner_ref[...]
    acc = jnp.zeros((1, 1), jnp.float32)
    for s in range(NSTRIP):
        r = slice(s * SR, (s + 1) * SR)
        rid = rowi + (s * SR)
        a = jnp.min(jnp.where(owner == rid, colj, K), axis=1, keepdims=True)
        nc = ncost_ref[r, :]
        cmax = jnp.max(nc, axis=1, keepdims=True)
        jmin = jnp.min(jnp.where(nc == cmax, colj, K), axis=1, keepdims=True)
        a = jnp.where(a == K, jmin, a)
        mask = colj == a
        mx = jnp.sum(jnp.where(mask, gx, 0.0), axis=1, keepdims=True)
        my = jnp.sum(jnp.where(mask, gy, 0.0), axis=1, keepdims=True)
        mz = jnp.sum(jnp.where(mask, gz, 0.0), axis=1, keepdims=True)
        cx = coarse_ref[r, 0:1]
        cy = coarse_ref[r, 1:2]
        cz = coarse_ref[r, 2:3]
        dx = cx - mx
        dy = cy - my
        dz = cz - mz
        dist = (dx * dx + dy * dy) + dz * dz
        acc = acc + jnp.sum(jnp.sqrt(jnp.maximum(dist, 1e-12)))
    out_ref[...] = acc


def _run_emd(coarse, gt_ds3):
    return pl.pallas_call(
        _emd_body,
        grid=(B,),
        in_specs=[
            pl.BlockSpec((None, K, 3), lambda b: (b, 0, 0)),
            pl.BlockSpec((None, 3, K), lambda b: (b, 0, 0)),
        ],
        out_specs=pl.BlockSpec((None, 1, 1), lambda b: (b, 0, 0)),
        out_shape=jax.ShapeDtypeStruct((B, 1, 1), jnp.float32),
        scratch_shapes=[
            pltpu.VMEM((K, K), jnp.float32),
            pltpu.VMEM((NSTRIP, K), jnp.float32),
            pltpu.VMEM((NSTRIP, K), jnp.int32),
            pltpu.VMEM((1, K), jnp.float32),
            pltpu.VMEM((1, K), jnp.int32),
        ],
        compiler_params=pltpu.CompilerParams(
            dimension_semantics=("parallel",)),
    )(coarse, gt_ds3)


CSR = 16
NCSTRIP = NF // CSR


def _chamfer_body(fine_ref, gt_ref, out_ref):
    gx = gt_ref[0:1, :]
    gy = gt_ref[1:2, :]
    gz = gt_ref[2:3, :]
    cm = jnp.full((1, NG), 1e30, jnp.float32)
    acc = jnp.zeros((1, 1), jnp.float32)
    for s in range(NCSTRIP):
        r = slice(s * CSR, (s + 1) * CSR)
        fx = fine_ref[r, 0:1]
        fy = fine_ref[r, 1:2]
        fz = fine_ref[r, 2:3]
        dx = fx - gx
        dy = fy - gy
        dz = fz - gz
        d2 = (dx * dx + dy * dy) + dz * dz
        acc = acc + jnp.sum(jnp.min(d2, axis=1))
        cm = jnp.minimum(cm, jnp.min(d2, axis=0, keepdims=True))
    out_ref[...] = acc / NF + jnp.sum(cm) / NG


def _run_chamfer(fine, gt3):
    return pl.pallas_call(
        _chamfer_body,
        grid=(B,),
        in_specs=[
            pl.BlockSpec((None, NF, 3), lambda b: (b, 0, 0)),
            pl.BlockSpec((None, 3, NG), lambda b: (b, 0, 0)),
        ],
        out_specs=pl.BlockSpec((None, 1, 1), lambda b: (b, 0, 0)),
        out_shape=jax.ShapeDtypeStruct((B, 1, 1), jnp.float32),
        compiler_params=pltpu.CompilerParams(
            dimension_semantics=("parallel",)),
    )(fine, gt3)


def kernel(coarse, fine, gt, alpha):
    gt3 = jnp.transpose(gt, (1, 0, 2))
    sel = _run_fps(gt3)

    gt_pad = jnp.pad(jnp.transpose(gt, (0, 2, 1)), ((0, 0), (0, 0), (0, 125)))
    sel3 = sel.reshape(B, NCHUNK, 128)
    flat = _run_scatter_sc(gt_pad, sel3)
    gt_dsb = jnp.transpose(
        flat.reshape(B, KP, 128)[:, :K, :3], (0, 2, 1))

    sums = _run_emd(coarse, gt_dsb)
    cham = _run_chamfer(fine, gt)

    loss_coarse = jnp.sum(sums) / (B * K)
    loss_fine = jnp.mean(cham)
    loss = loss_coarse + alpha * loss_fine
    return (loss, loss_coarse, loss_fine)


# speedup vs baseline: 34.4253x; 1.0054x over previous
import functools

import jax
import jax.numpy as jnp
from jax import lax
from jax.experimental import pallas as pl
from jax.experimental.pallas import tpu as pltpu
from jax.experimental.pallas import tpu_sc as plsc

B = 8
K = 1024
NF = 2048
NG = 4096
NEG = -1e10
HALF_NEG = -5e9
EPS = 0.005
ITERS = 50
SR = 64
NSTRIP = K // SR


def _fps_body(gt_ref, sel_ref):
    gx = gt_ref[0]
    gy = gt_ref[1]
    gz = gt_ref[2]
    lane = jax.lax.broadcasted_iota(jnp.int32, (B, NG), 1)
    sel_ref[...] = lane * 0 + K

    def step(k, carry):
        last, dists = carry
        mask2 = lane == last
        sel_ref[...] = jnp.where(mask2, k, sel_ref[...])
        lx = jnp.sum(jnp.where(mask2, gx, 0.0), axis=1, keepdims=True)
        ly = jnp.sum(jnp.where(mask2, gy, 0.0), axis=1, keepdims=True)
        lz = jnp.sum(jnp.where(mask2, gz, 0.0), axis=1, keepdims=True)
        dx = gx - lx
        dy = gy - ly
        dz = gz - lz
        d = (dx * dx + dy * dy) + dz * dz
        dists = jnp.minimum(dists, d)
        m = jnp.max(dists, axis=1, keepdims=True)
        nxt = jnp.min(jnp.where(dists == m, lane, NG), axis=1, keepdims=True)
        return nxt, dists

    init = (jnp.zeros((B, 1), jnp.int32), gx * 0.0 + 1e10)
    jax.lax.fori_loop(0, K, step, init)


def _run_fps(gt3):
    return pl.pallas_call(
        _fps_body,
        out_shape=jax.ShapeDtypeStruct((B, NG), jnp.int32),
    )(gt3)


KP = K + 8
NCHUNK = NG // 128


def _run_scatter_sc(gt_pad, sel3):
    info = plsc.get_sparse_core_info()
    nc, ns, nl = info.num_cores, info.num_subcores, info.num_lanes

    @functools.partial(
        pl.kernel,
        mesh=plsc.VectorSubcoreMesh(core_axis_name="c", subcore_axis_name="s"),
        out_type=jax.ShapeDtypeStruct((B * KP, 128), jnp.float32),
        scratch_types=[
            pltpu.VMEM((NCHUNK, 128), jnp.int32),
            pltpu.VMEM((128, 128), jnp.float32),
            pltpu.SemaphoreType.DMA,
        ],
    )
    def k(gt_hbm, sel_hbm, out_hbm, idx_v, rows_v, sem):
        wid = lax.axis_index("s") * nc + lax.axis_index("c")

        @pl.when(wid < B)
        def _():
            b = wid
            pltpu.sync_copy(sel_hbm.at[b], idx_v)
            off = b * KP

            def obody(i, carry):
                r = i // (128 // nl)
                g = i % (128 // nl)
                s = pl.ds(g * nl, nl)
                idx_v[r, s] = idx_v[r, s] + off
                return carry

            lax.fori_loop(0, NCHUNK * (128 // nl), obody, jnp.int32(0))

            def body(j, carry):
                pltpu.sync_copy(gt_hbm.at[b, pl.ds(j * 128, 128)], rows_v)
                pltpu.async_copy(
                    rows_v,
                    out_hbm.at[idx_v.at[j]],
                    sem,
                ).wait()
                return carry

            lax.fori_loop(0, NCHUNK, body, jnp.int32(0))

    return k(gt_pad, sel3)


def _emd_body(coarse_ref, y_ref, out_ref, ncost_ref, pmax_ref, parg_ref,
              price_ref, owner_ref):
    gx = y_ref[0:1, :]
    gy = y_ref[1:2, :]
    gz = y_ref[2:3, :]
    colj = jax.lax.broadcasted_iota(jnp.int32, (SR, K), 1)
    rowi = jax.lax.broadcasted_iota(jnp.int32, (SR, 1), 0)

    for s in range(NSTRIP):
        r = slice(s * SR, (s + 1) * SR)
        cx = coarse_ref[r, 0:1]
        cy = coarse_ref[r, 1:2]
        cz = coarse_ref[r, 2:3]
        dx = cx - gx
        dy = cy - gy
        dz = cz - gz
        ncost_ref[r, :] = -((dx * dx + dy * dy) + dz * dz)
    price_ref[...] = jnp.zeros((1, K), jnp.float32)
    owner_ref[...] = jnp.full((1, K), -1, jnp.int32)

    def auction_iter(_, carry):
        price = price_ref[...]
        owner = owner_ref[...]
        for s in range(NSTRIP):
            r = slice(s * SR, (s + 1) * SR)
            rid = rowi + (s * SR)
            unass = ~jnp.any(owner == rid, axis=1, keepdims=True)
            value = ncost_ref[r, :] - price
            m1 = jnp.max(value, axis=1, keepdims=True)
            i1 = jnp.min(jnp.where(value == m1, colj, K), axis=1,
                         keepdims=True)
            mask = colj == i1
            v2 = jnp.max(jnp.where(mask, NEG, value), axis=1, keepdims=True)
            bid_m = jnp.where(unass, (m1 - v2) + EPS, NEG)
            contrib = jnp.where(mask, bid_m, NEG)
            pm = jnp.max(contrib, axis=0, keepdims=True)
            pa = jnp.min(
                jnp.where((contrib == pm) & (pm > HALF_NEG), rid, K),
                axis=0, keepdims=True)
            pmax_ref[s:s + 1, :] = pm
            parg_ref[s:s + 1, :] = pa
        pms = pmax_ref[...]
        hd = jnp.max(pms, axis=0, keepdims=True)
        winner = jnp.min(
            jnp.where(pms == hd, parg_ref[...], K), axis=0, keepdims=True)
        has_w = winner < K
        wc = jnp.minimum(winner, K - 1)
        price_ref[...] = jnp.where(has_w, price + hd, price)
        owner_ref[...] = jnp.where(has_w, wc, owner)
        return carry

    jax.lax.fori_loop(0, ITERS, auction_iter, jnp.int32(0))

    owner = owner_ref[...]
    acc = jnp.zeros((1, 1), jnp.float32)
    for s in range(NSTRIP):
        r = slice(s * SR, (s + 1) * SR)
        rid = rowi + (s * SR)
        a = jnp.min(jnp.where(owner == rid, colj, K), axis=1, keepdims=True)
        nc = ncost_ref[r, :]
        cmax = jnp.max(nc, axis=1, keepdims=True)
        jmin = jnp.min(jnp.where(nc == cmax, colj, K), axis=1, keepdims=True)
        a = jnp.where(a == K, jmin, a)
        mask = colj == a
        mx = jnp.sum(jnp.where(mask, gx, 0.0), axis=1, keepdims=True)
        my = jnp.sum(jnp.where(mask, gy, 0.0), axis=1, keepdims=True)
        mz = jnp.sum(jnp.where(mask, gz, 0.0), axis=1, keepdims=True)
        cx = coarse_ref[r, 0:1]
        cy = coarse_ref[r, 1:2]
        cz = coarse_ref[r, 2:3]
        dx = cx - mx
        dy = cy - my
        dz = cz - mz
        dist = (dx * dx + dy * dy) + dz * dz
        acc = acc + jnp.sum(jnp.sqrt(jnp.maximum(dist, 1e-12)))
    out_ref[...] = acc


def _run_emd(coarse, gt_ds3):
    return pl.pallas_call(
        _emd_body,
        grid=(B,),
        in_specs=[
            pl.BlockSpec((None, K, 3), lambda b: (b, 0, 0)),
            pl.BlockSpec((None, 3, K), lambda b: (b, 0, 0)),
        ],
        out_specs=pl.BlockSpec((None, 1, 1), lambda b: (b, 0, 0)),
        out_shape=jax.ShapeDtypeStruct((B, 1, 1), jnp.float32),
        scratch_shapes=[
            pltpu.VMEM((K, K), jnp.float32),
            pltpu.VMEM((NSTRIP, K), jnp.float32),
            pltpu.VMEM((NSTRIP, K), jnp.int32),
            pltpu.VMEM((1, K), jnp.float32),
            pltpu.VMEM((1, K), jnp.int32),
        ],
        compiler_params=pltpu.CompilerParams(
            dimension_semantics=("parallel",)),
    )(coarse, gt_ds3)


CSR = 32
NCSTRIP = NF // CSR


def _chamfer_body(fine_ref, gt_ref, out_ref):
    g = gt_ref[...]
    gx = gt_ref[0:1, :]
    gy = gt_ref[1:2, :]
    gz = gt_ref[2:3, :]
    gn = (gx * gx + gy * gy) + gz * gz
    cm = jnp.full((1, NG), 1e30, jnp.float32)
    acc = jnp.zeros((1, 1), jnp.float32)
    for s in range(NCSTRIP):
        f = fine_ref[s * CSR:(s + 1) * CSR, :]
        fn = jnp.sum(f * f, axis=1, keepdims=True)
        m = jax.lax.dot_general(f, g, (((1,), (0,)), ((), ())),
                                precision=jax.lax.Precision.HIGHEST,
                                preferred_element_type=jnp.float32)
        d2 = (fn + gn) - (m + m)
        acc = acc + jnp.sum(jnp.min(d2, axis=1))
        cm = jnp.minimum(cm, jnp.min(d2, axis=0, keepdims=True))
    out_ref[...] = acc / NF + jnp.sum(cm) / NG


def _run_chamfer(fine, gt3):
    return pl.pallas_call(
        _chamfer_body,
        grid=(B,),
        in_specs=[
            pl.BlockSpec((None, NF, 3), lambda b: (b, 0, 0)),
            pl.BlockSpec((None, 3, NG), lambda b: (b, 0, 0)),
        ],
        out_specs=pl.BlockSpec((None, 1, 1), lambda b: (b, 0, 0)),
        out_shape=jax.ShapeDtypeStruct((B, 1, 1), jnp.float32),
        compiler_params=pltpu.CompilerParams(
            dimension_semantics=("parallel",)),
    )(fine, gt3)


def kernel(coarse, fine, gt, alpha):
    gt3 = jnp.transpose(gt, (1, 0, 2))
    sel = _run_fps(gt3)

    gt_pad = jnp.pad(jnp.transpose(gt, (0, 2, 1)), ((0, 0), (0, 0), (0, 125)))
    sel3 = sel.reshape(B, NCHUNK, 128)
    flat = _run_scatter_sc(gt_pad, sel3)
    gt_dsb = jnp.transpose(
        flat.reshape(B, KP, 128)[:, :K, :3], (0, 2, 1))

    sums = _run_emd(coarse, gt_dsb)
    cham = _run_chamfer(fine, gt)

    loss_coarse = jnp.sum(sums) / (B * K)
    loss_fine = jnp.mean(cham)
    loss = loss_coarse + alpha * loss_fine
    return (loss, loss_coarse, loss_fine)


# speedup vs baseline: 42.2757x; 1.2280x over previous
import functools

import jax
import jax.numpy as jnp
from jax import lax
from jax.experimental import pallas as pl
from jax.experimental.pallas import tpu as pltpu
from jax.experimental.pallas import tpu_sc as plsc

B = 8
K = 1024
NF = 2048
NG = 4096
NEG = -1e10
HALF_NEG = -5e9
EPS = 0.005
ITERS = 50
SR = 64
NSTRIP = K // SR


def _fps_body(gt_ref, sel_ref):
    gx = gt_ref[0]
    gy = gt_ref[1]
    gz = gt_ref[2]
    lane_i = jax.lax.broadcasted_iota(jnp.int32, (B, NG), 1)
    lane = lane_i.astype(jnp.float32)
    sel_ref[...] = lane_i * 0 + K

    def step(k, carry):
        last, dists = carry
        mask2 = lane == last
        sel_ref[...] = jnp.where(mask2, k, sel_ref[...])
        lx = jnp.sum(jnp.where(mask2, gx, 0.0), axis=1, keepdims=True)
        ly = jnp.sum(jnp.where(mask2, gy, 0.0), axis=1, keepdims=True)
        lz = jnp.sum(jnp.where(mask2, gz, 0.0), axis=1, keepdims=True)
        dx = gx - lx
        dy = gy - ly
        dz = gz - lz
        d = (dx * dx + dy * dy) + dz * dz
        dists = jnp.minimum(dists, d)
        m = jnp.max(dists, axis=1, keepdims=True)
        nxt = jnp.min(jnp.where(dists == m, lane, float(NG)), axis=1,
                      keepdims=True)
        return nxt, dists

    init = (jnp.zeros((B, 1), jnp.float32), gx * 0.0 + 1e10)
    jax.lax.fori_loop(0, K, step, init)


def _run_fps(gt3):
    return pl.pallas_call(
        _fps_body,
        out_shape=jax.ShapeDtypeStruct((B, NG), jnp.int32),
    )(gt3)


KP = K + 8
NCHUNK = NG // 128


def _run_scatter_sc(gt_pad, sel3):
    info = plsc.get_sparse_core_info()
    nc, ns, nl = info.num_cores, info.num_subcores, info.num_lanes

    @functools.partial(
        pl.kernel,
        mesh=plsc.VectorSubcoreMesh(core_axis_name="c", subcore_axis_name="s"),
        out_type=jax.ShapeDtypeStruct((B * KP, 128), jnp.float32),
        scratch_types=[
            pltpu.VMEM((NCHUNK, 128), jnp.int32),
            pltpu.VMEM((128, 128), jnp.float32),
            pltpu.SemaphoreType.DMA,
        ],
    )
    def k(gt_hbm, sel_hbm, out_hbm, idx_v, rows_v, sem):
        wid = lax.axis_index("s") * nc + lax.axis_index("c")

        @pl.when(wid < B)
        def _():
            b = wid
            pltpu.sync_copy(sel_hbm.at[b], idx_v)
            off = b * KP

            def obody(i, carry):
                r = i // (128 // nl)
                g = i % (128 // nl)
                s = pl.ds(g * nl, nl)
                idx_v[r, s] = idx_v[r, s] + off
                return carry

            lax.fori_loop(0, NCHUNK * (128 // nl), obody, jnp.int32(0))

            def body(j, carry):
                pltpu.sync_copy(gt_hbm.at[b, pl.ds(j * 128, 128)], rows_v)
                pltpu.async_copy(
                    rows_v,
                    out_hbm.at[idx_v.at[j]],
                    sem,
                ).wait()
                return carry

            lax.fori_loop(0, NCHUNK, body, jnp.int32(0))

    return k(gt_pad, sel3)


def _emd_body(coarse_ref, y_ref, out_ref, ncost_ref, pmax_ref, parg_ref,
              price_ref, owner_ref):
    gx = y_ref[0:1, :]
    gy = y_ref[1:2, :]
    gz = y_ref[2:3, :]
    colj = jax.lax.broadcasted_iota(jnp.int32, (SR, K), 1).astype(jnp.float32)
    rowi = jax.lax.broadcasted_iota(jnp.int32, (SR, 1), 0).astype(jnp.float32)
    KF = float(K)

    for s in range(NSTRIP):
        r = slice(s * SR, (s + 1) * SR)
        cx = coarse_ref[r, 0:1]
        cy = coarse_ref[r, 1:2]
        cz = coarse_ref[r, 2:3]
        dx = cx - gx
        dy = cy - gy
        dz = cz - gz
        ncost_ref[r, :] = -((dx * dx + dy * dy) + dz * dz)
    price_ref[...] = jnp.zeros((1, K), jnp.float32)
    owner_ref[...] = jnp.full((1, K), -1.0, jnp.float32)

    def auction_iter(_, carry):
        price = price_ref[...]
        owner = owner_ref[...]
        for s in range(NSTRIP):
            r = slice(s * SR, (s + 1) * SR)
            rid = rowi + (s * SR)
            unass = ~jnp.any(owner == rid, axis=1, keepdims=True)
            value = ncost_ref[r, :] - price
            m1 = jnp.max(value, axis=1, keepdims=True)
            i1 = jnp.min(jnp.where(value == m1, colj, KF), axis=1,
                         keepdims=True)
            mask = colj == i1
            v2 = jnp.max(jnp.where(mask, NEG, value), axis=1, keepdims=True)
            bid_m = jnp.where(unass, (m1 - v2) + EPS, NEG)
            contrib = jnp.where(mask, bid_m, NEG)
            pm = jnp.max(contrib, axis=0, keepdims=True)
            pa = jnp.min(
                jnp.where((contrib == pm) & (pm > HALF_NEG), rid, KF),
                axis=0, keepdims=True)
            pmax_ref[s:s + 1, :] = pm
            parg_ref[s:s + 1, :] = pa
        pms = pmax_ref[...]
        hd = jnp.max(pms, axis=0, keepdims=True)
        winner = jnp.min(
            jnp.where(pms == hd, parg_ref[...], KF), axis=0, keepdims=True)
        has_w = winner < KF
        wc = jnp.minimum(winner, KF - 1.0)
        price_ref[...] = jnp.where(has_w, price + hd, price)
        owner_ref[...] = jnp.where(has_w, wc, owner)
        return carry

    jax.lax.fori_loop(0, ITERS, auction_iter, jnp.int32(0))

    owner = owner_ref[...]
    acc = jnp.zeros((1, 1), jnp.float32)
    for s in range(NSTRIP):
        r = slice(s * SR, (s + 1) * SR)
        rid = rowi + (s * SR)
        a = jnp.min(jnp.where(owner == rid, colj, KF), axis=1, keepdims=True)
        nc = ncost_ref[r, :]
        cmax = jnp.max(nc, axis=1, keepdims=True)
        jmin = jnp.min(jnp.where(nc == cmax, colj, KF), axis=1, keepdims=True)
        a = jnp.where(a == KF, jmin, a)
        mask = colj == a
        mx = jnp.sum(jnp.where(mask, gx, 0.0), axis=1, keepdims=True)
        my = jnp.sum(jnp.where(mask, gy, 0.0), axis=1, keepdims=True)
        mz = jnp.sum(jnp.where(mask, gz, 0.0), axis=1, keepdims=True)
        cx = coarse_ref[r, 0:1]
        cy = coarse_ref[r, 1:2]
        cz = coarse_ref[r, 2:3]
        dx = cx - mx
        dy = cy - my
        dz = cz - mz
        dist = (dx * dx + dy * dy) + dz * dz
        acc = acc + jnp.sum(jnp.sqrt(jnp.maximum(dist, 1e-12)))
    out_ref[...] = acc


def _run_emd(coarse, gt_ds3):
    return pl.pallas_call(
        _emd_body,
        grid=(B,),
        in_specs=[
            pl.BlockSpec((None, K, 3), lambda b: (b, 0, 0)),
            pl.BlockSpec((None, 3, K), lambda b: (b, 0, 0)),
        ],
        out_specs=pl.BlockSpec((None, 1, 1), lambda b: (b, 0, 0)),
        out_shape=jax.ShapeDtypeStruct((B, 1, 1), jnp.float32),
        scratch_shapes=[
            pltpu.VMEM((K, K), jnp.float32),
            pltpu.VMEM((NSTRIP, K), jnp.float32),
            pltpu.VMEM((NSTRIP, K), jnp.float32),
            pltpu.VMEM((1, K), jnp.float32),
            pltpu.VMEM((1, K), jnp.float32),
        ],
        compiler_params=pltpu.CompilerParams(
            dimension_semantics=("parallel",)),
    )(coarse, gt_ds3)


CSR = 32
NCSTRIP = NF // CSR


def _chamfer_body(fine_ref, gt_ref, out_ref):
    g = gt_ref[...]
    gx = gt_ref[0:1, :]
    gy = gt_ref[1:2, :]
    gz = gt_ref[2:3, :]
    gn = (gx * gx + gy * gy) + gz * gz
    cm = jnp.full((1, NG), 1e30, jnp.float32)
    acc = jnp.zeros((1, 1), jnp.float32)
    for s in range(NCSTRIP):
        f = fine_ref[s * CSR:(s + 1) * CSR, :]
        fn = jnp.sum(f * f, axis=1, keepdims=True)
        m = jax.lax.dot_general(f, g, (((1,), (0,)), ((), ())),
                                precision=jax.lax.Precision.HIGHEST,
                                preferred_element_type=jnp.float32)
        d2 = (fn + gn) - (m + m)
        acc = acc + jnp.sum(jnp.min(d2, axis=1))
        cm = jnp.minimum(cm, jnp.min(d2, axis=0, keepdims=True))
    out_ref[...] = acc / NF + jnp.sum(cm) / NG


def _run_chamfer(fine, gt3):
    return pl.pallas_call(
        _chamfer_body,
        grid=(B,),
        in_specs=[
            pl.BlockSpec((None, NF, 3), lambda b: (b, 0, 0)),
            pl.BlockSpec((None, 3, NG), lambda b: (b, 0, 0)),
        ],
        out_specs=pl.BlockSpec((None, 1, 1), lambda b: (b, 0, 0)),
        out_shape=jax.ShapeDtypeStruct((B, 1, 1), jnp.float32),
        compiler_params=pltpu.CompilerParams(
            dimension_semantics=("parallel",)),
    )(fine, gt3)


def kernel(coarse, fine, gt, alpha):
    gt3 = jnp.transpose(gt, (1, 0, 2))
    sel = _run_fps(gt3)

    gt_pad = jnp.pad(jnp.transpose(gt, (0, 2, 1)), ((0, 0), (0, 0), (0, 125)))
    sel3 = sel.reshape(B, NCHUNK, 128)
    flat = _run_scatter_sc(gt_pad, sel3)
    gt_dsb = jnp.transpose(
        flat.reshape(B, KP, 128)[:, :K, :3], (0, 2, 1))

    sums = _run_emd(coarse, gt_dsb)
    cham = _run_chamfer(fine, gt)

    loss_coarse = jnp.sum(sums) / (B * K)
    loss_fine = jnp.mean(cham)
    loss = loss_coarse + alpha * loss_fine
    return (loss, loss_coarse, loss_fine)
